# Initial kernel scaffold; baseline (speedup 1.0000x reference)
import jax
import jax.numpy as jnp
from jax.experimental import pallas as pl


def kernel(points, template_points, W_mean, b_mean, W_dec, b_dec):
    raise NotImplementedError("write your pallas kernel here")


# speedup vs baseline: 197.1584x; 197.1584x over previous
import jax
import jax.numpy as jnp
from jax import lax
from jax.experimental import pallas as pl
from jax.experimental.pallas import tpu as pltpu
from jax.experimental.pallas import tpu_sc as plsc

B = 16
N = 50000
GS = 16
NT = 6
DT = 1.0 / (NT - 1)
SCALE = (GS - 1) / 6.0
OFF = 3.0 * SCALE
L0 = 26000
L1 = N - L0
NV0 = L0 // 16
NV1 = L1 // 16

_MESH = plsc.VectorSubcoreMesh(core_axis_name="c", subcore_axis_name="s",
                               num_cores=2, num_subcores=16)
_SC_PARAMS = pltpu.CompilerParams(needs_layout_passes=False)


def _splat_body(pts_hbm, tpl_hbm, out_hbm, pbuf, tbuf, gridx, gridy, obuf):
    core = lax.axis_index("c")
    sub = lax.axis_index("s")
    pbase = pl.multiple_of(sub * (2 * N), 8)

    zero16 = jnp.zeros((16,), jnp.float32)

    def zbody(k, carry):
        gridx[pl.ds(k * 16, 16)] = zero16
        gridy[pl.ds(k * 16, 16)] = zero16
        return carry

    lax.fori_loop(0, 256, zbody, 0)

    @pl.when(core == 0)
    def _():
        pltpu.sync_copy(pts_hbm.at[pl.ds(pbase, 2 * L0)], pbuf.at[pl.ds(0, 2 * L0)])
        pltpu.sync_copy(tpl_hbm.at[pl.ds(0, 2 * L0)], tbuf.at[pl.ds(0, 2 * L0)])

    @pl.when(core == 1)
    def _():
        pltpu.sync_copy(pts_hbm.at[pl.ds(pbase + 2 * L0, 2 * L1)],
                        pbuf.at[pl.ds(0, 2 * L1)])
        pltpu.sync_copy(tpl_hbm.at[pl.ds(2 * L0, 2 * L1)], tbuf.at[pl.ds(0, 2 * L1)])

    nv = jnp.where(core == 0, NV0, NV1)
    iota = lax.iota(jnp.int32, 16)
    iota2 = iota * 2
    lanebase = iota * 256

    def body(r, carry):
        xpos = iota2 + r * 32
        ypos = xpos + 1
        x = plsc.load_gather(pbuf, [xpos])
        y = plsc.load_gather(pbuf, [ypos])
        tx = plsc.load_gather(tbuf, [xpos])
        ty = plsc.load_gather(tbuf, [ypos])
        u = x * SCALE + OFF
        v = y * SCALE + OFF
        iu = u.astype(jnp.int32)
        iv = v.astype(jnp.int32)
        iu1 = jnp.minimum(jnp.maximum(iu, 0), GS - 1)
        iv1 = jnp.minimum(jnp.maximum(iv, 0), GS - 1)
        iu2 = jnp.minimum(iu1 + 1, GS - 1)
        iv2 = jnp.minimum(iv1 + 1, GS - 1)
        fu = u - iu1.astype(jnp.float32)
        fv = v - iv1.astype(jnp.float32)
        gu = 1.0 - fu
        gv = 1.0 - fv
        s1 = iu1 * 16 + lanebase
        s2 = iu2 * 16 + lanebase
        c11 = s1 + iv1
        c12 = s1 + iv2
        c21 = s2 + iv1
        c22 = s2 + iv2
        vx = x - tx
        vy = y - ty
        w11 = gu * gv
        w12 = gu * fv
        w21 = fu * gv
        w22 = fu * fv
        plsc.addupdate_scatter(gridx, [c11], vx * w11)
        plsc.addupdate_scatter(gridx, [c12], vx * w12)
        plsc.addupdate_scatter(gridx, [c21], vx * w21)
        plsc.addupdate_scatter(gridx, [c22], vx * w22)
        plsc.addupdate_scatter(gridy, [c11], vy * w11)
        plsc.addupdate_scatter(gridy, [c12], vy * w12)
        plsc.addupdate_scatter(gridy, [c21], vy * w21)
        plsc.addupdate_scatter(gridy, [c22], vy * w22)
        return carry

    lax.fori_loop(0, nv, body, 0)

    for ch, grid in ((0, gridx), (1, gridy)):
        for j in range(16):
            acc = grid[pl.ds(j * 16, 16)]
            for l in range(1, 16):
                acc = acc + grid[pl.ds(l * 256 + j * 16, 16)]
            obuf[pl.ds(ch * 256 + j * 16, 16)] = acc

    obase = pl.multiple_of(core * (B * 512) + sub * 512, 8)
    pltpu.sync_copy(obuf, out_hbm.at[pl.ds(obase, 512)])


_splat = pl.kernel(
    _splat_body,
    out_type=jax.ShapeDtypeStruct((2 * B * 512,), jnp.float32),
    mesh=_MESH,
    compiler_params=_SC_PARAMS,
    scratch_types=[
        pltpu.VMEM((2 * L0,), jnp.float32),
        pltpu.VMEM((2 * L0,), jnp.float32),
        pltpu.VMEM((16 * 256,), jnp.float32),
        pltpu.VMEM((16 * 256,), jnp.float32),
        pltpu.VMEM((512,), jnp.float32),
    ],
)


def _dense_body(part_ref, wm_ref, bm_ref, wd_ref, bd_ref, out_ref):
    hp = lax.Precision.HIGHEST
    grid = part_ref[0] + part_ref[1]
    means = jnp.dot(grid.astype(jnp.bfloat16),
                    wm_ref[...].astype(jnp.bfloat16),
                    preferred_element_type=jnp.float32) + bm_ref[...]
    act = jnp.dot(means.astype(jnp.bfloat16),
                  wd_ref[...].astype(jnp.bfloat16),
                  preferred_element_type=jnp.float32) + bd_ref[...]
    vel = jnp.tanh(act)

    r = lax.broadcasted_iota(jnp.int32, (512, 512), 0)
    c = lax.broadcasted_iota(jnp.int32, (512, 512), 1)
    jj = c % 16
    ii = (c // 16) % 16
    pj = jnp.where(jj < 15, c + 1, c)
    pi = jnp.where(ii < 15, c + 16, c)
    pij = jnp.where(ii < 15, pj + 16, pj)
    sj = (r == pj).astype(jnp.float32)
    si = (r == pi).astype(jnp.float32)
    sij = (r == pij).astype(jnp.float32)
    velj = jnp.dot(vel, sj, precision=hp, preferred_element_type=jnp.float32)
    veli = jnp.dot(vel, si, precision=hp, preferred_element_type=jnp.float32)
    velij = jnp.dot(vel, sij, precision=hp, preferred_element_type=jnp.float32)
    c0 = DT * vel
    c1 = DT * (veli - vel)
    c2 = DT * (velj - vel)
    c3 = DT * (velij - veli - velj + vel)
    out_ref[...] = jnp.stack([c0, c1, c2, c3], axis=1)


def _dense(partial, wm, bm, wd, bd):
    return pl.pallas_call(
        _dense_body,
        out_shape=jax.ShapeDtypeStruct((B, 4, 512), jnp.float32),
    )(partial.reshape(2, B, 512), wm, bm, wd, bd)


def _shoot_body(tpl_hbm, tab_hbm, out_hbm, tbuf, obuf, *cbufs):
    core = lax.axis_index("c")
    sub = lax.axis_index("s")
    tbase = pl.multiple_of(sub * 2048, 8)
    obase = pl.multiple_of(sub * (2 * N), 8)

    for t in range(4):
        for ch in range(2):
            pltpu.sync_copy(tab_hbm.at[pl.ds(tbase + t * 512 + ch * 256, 256)],
                            cbufs[2 * t + ch])

    @pl.when(core == 0)
    def _():
        pltpu.sync_copy(tpl_hbm.at[pl.ds(0, 2 * L0)], tbuf.at[pl.ds(0, 2 * L0)])

    @pl.when(core == 1)
    def _():
        pltpu.sync_copy(tpl_hbm.at[pl.ds(2 * L0, 2 * L1)], tbuf.at[pl.ds(0, 2 * L1)])

    nv = jnp.where(core == 0, NV0, NV1)
    iota = lax.iota(jnp.int32, 16)
    iota2 = iota * 2

    def body(r, carry):
        xpos = iota2 + r * 32
        ypos = xpos + 1
        x = plsc.load_gather(tbuf, [xpos])
        y = plsc.load_gather(tbuf, [ypos])
        for _step in range(NT - 1):
            u = x * SCALE + OFF
            v = y * SCALE + OFF
            iu = u.astype(jnp.int32)
            iv = v.astype(jnp.int32)
            iu1 = jnp.minimum(jnp.maximum(iu, 0), GS - 1)
            iv1 = jnp.minimum(jnp.maximum(iv, 0), GS - 1)
            fu = u - iu1.astype(jnp.float32)
            fv = v - iv1.astype(jnp.float32)
            cell = iu1 * 16 + iv1
            t = fu * fv
            c0x = plsc.load_gather(cbufs[0], [cell])
            c0y = plsc.load_gather(cbufs[1], [cell])
            c1x = plsc.load_gather(cbufs[2], [cell])
            c1y = plsc.load_gather(cbufs[3], [cell])
            c2x = plsc.load_gather(cbufs[4], [cell])
            c2y = plsc.load_gather(cbufs[5], [cell])
            c3x = plsc.load_gather(cbufs[6], [cell])
            c3y = plsc.load_gather(cbufs[7], [cell])
            x = x + (c0x + fu * c1x + fv * c2x + t * c3x)
            y = y + (c0y + fu * c1y + fv * c2y + t * c3y)
        plsc.store_scatter(obuf, [xpos], x)
        plsc.store_scatter(obuf, [ypos], y)
        return carry

    lax.fori_loop(0, nv, body, 0)

    @pl.when(core == 0)
    def _():
        pltpu.sync_copy(obuf.at[pl.ds(0, 2 * L0)], out_hbm.at[pl.ds(obase, 2 * L0)])

    @pl.when(core == 1)
    def _():
        pltpu.sync_copy(obuf.at[pl.ds(0, 2 * L1)],
                        out_hbm.at[pl.ds(obase + 2 * L0, 2 * L1)])


_shoot = pl.kernel(
    _shoot_body,
    out_type=jax.ShapeDtypeStruct((B * 2 * N,), jnp.float32),
    mesh=_MESH,
    compiler_params=_SC_PARAMS,
    scratch_types=[
        pltpu.VMEM((2 * L0,), jnp.float32),
        pltpu.VMEM((2 * L0,), jnp.float32),
    ] + [pltpu.VMEM((256,), jnp.float32) for _ in range(8)],
)


@jax.jit
def kernel(points, template_points, W_mean, b_mean, W_dec, b_dec):
    wm = jnp.pad(W_mean, ((0, 0), (0, 2)))
    bm = jnp.pad(b_mean, (0, 2))
    wd = jnp.pad(W_dec, ((0, 2), (0, 0)))
    pts_flat = points.reshape(B * 2 * N)
    tpl_flat = template_points.reshape(2 * N)
    partial = _splat(pts_flat, tpl_flat)
    tables = _dense(partial, wm, bm, wd, b_dec)
    out = _shoot(tpl_flat, tables.reshape(B * 4 * 512))
    return out.reshape(B, N, 2)
